# baseline (speedup 1.0000x reference)
import functools

import jax
import jax.numpy as jnp
from jax import lax
from jax.experimental import pallas as pl
from jax.experimental.pallas import tpu as pltpu
from jax.experimental.pallas import tpu_sc as plsc

TOP_K = 8
ALPHA = 0.001


def _gate_body(seq, block, n_experts, batch, const,
               hs_a_ref, hs_b_ref, w_ref, idx_ref, wt_ref, ss_ref, sc_acc):
    i = pl.program_id(0)
    nb = pl.num_programs(0)
    bpb = seq // block
    b = i // bpb

    @pl.when(i == 0)
    def _init():
        sc_acc[...] = jnp.zeros_like(sc_acc)

    w = w_ref[...]
    lt_a = jax.lax.dot_general(
        w, hs_a_ref[...], (((1,), (1,)), ((), ())),
        preferred_element_type=jnp.float32,
        precision=jax.lax.Precision.DEFAULT)
    lt_b = jax.lax.dot_general(
        w, hs_b_ref[...], (((1,), (1,)), ((), ())),
        preferred_element_type=jnp.float32,
        precision=jax.lax.Precision.DEFAULT)
    lt = jnp.concatenate([lt_a, lt_b], axis=1)

    m = jnp.max(lt, axis=0, keepdims=True)
    ex = jnp.exp(lt - m)
    s = jnp.sum(ex, axis=0, keepdims=True)

    rows = jax.lax.broadcasted_iota(jnp.int32, (n_experts, block), 0)
    rowsk = jax.lax.broadcasted_iota(jnp.int32, (TOP_K, block), 0)
    cur = lt
    mk8 = jnp.zeros((TOP_K, block), jnp.float32)
    ik8 = jnp.zeros((TOP_K, block), jnp.int32)
    for k in range(TOP_K):
        mk = jnp.max(cur, axis=0, keepdims=True)
        cand = jnp.where(cur == mk, rows, n_experts)
        ik = jnp.min(cand, axis=0, keepdims=True)
        mk8 = jnp.where(rowsk == k, mk, mk8)
        ik8 = jnp.where(rowsk == k, ik, ik8)
        cur = jnp.where(rows == ik, -jnp.inf, cur)
    w8 = jnp.exp(mk8 - m) / s
    idx_ref[...] = ik8.T
    wt_ref[...] = w8.T

    sc_local = jnp.sum(ex * (1.0 / s), axis=1, keepdims=True)
    lanes = jax.lax.broadcasted_iota(jnp.int32, sc_acc.shape, 1)
    bm = lanes == b
    sc_acc[...] = sc_acc[...] + jnp.where(bm, sc_local, 0.0)

    @pl.when(i == nb - 1)
    def _fin():
        ss_ref[...] = (sc_acc[...].T)[:batch, :] * const


def _make_sc_aux(n, n_experts, batch, seq):
    info = plsc.get_sparse_core_info()
    nw = info.num_cores * info.num_subcores
    per_w = n // nw
    per_wi = per_w * TOP_K
    chunks = per_wi // 16
    wpb = (batch * seq // per_w) // batch
    mesh = plsc.VectorSubcoreMesh(core_axis_name="c", subcore_axis_name="s")

    @functools.partial(
        pl.kernel, mesh=mesh,
        compiler_params=pltpu.CompilerParams(needs_layout_passes=False),
        out_type=jax.ShapeDtypeStruct((nw, 16), jnp.float32),
        scratch_types=[
            pltpu.VMEM((per_wi,), jnp.int32),
            pltpu.VMEM((2 * n_experts,), jnp.float32),
            pltpu.VMEM((n_experts,), jnp.float32),
            pltpu.VMEM((16,), jnp.float32),
        ],
    )
    def sc_aux(idx_hbm, ss_hbm, out_hbm, idx_v, cnt_v, row_v, acc_v):
        wid = lax.axis_index("s") * info.num_cores + lax.axis_index("c")
        b = wid // wpb
        pltpu.sync_copy(idx_hbm.at[pl.ds(wid * per_wi, per_wi)], idx_v)
        pltpu.sync_copy(ss_hbm.at[b], row_v)
        for j in range(2 * n_experts // 16):
            cnt_v[pl.ds(j * 16, 16)] = jnp.zeros((16,), jnp.float32)
        lane = jax.lax.broadcasted_iota(jnp.int32, (16,), 0)
        off = jnp.where(lane >= TOP_K, n_experts, 0)
        ones = jnp.ones((16,), jnp.float32)

        def body(j, _):
            v = idx_v[pl.ds(j * 16, 16)] + off
            plsc.addupdate_scatter(cnt_v, [v], ones)
            return 0

        lax.fori_loop(0, chunks, body, 0)
        acc = jnp.zeros((16,), jnp.float32)
        for j in range(2 * n_experts // 16):
            c = cnt_v[pl.ds(j * 16, 16)]
            r = row_v[pl.ds((j % (n_experts // 16)) * 16, 16)]
            acc = acc + c * r
        acc_v[...] = acc
        pltpu.sync_copy(acc_v, out_hbm.at[wid])

    return sc_aux


def _tc_call(hs, gate_weight, seq, block, n_experts, bsz, const,
             interpret=False):
    n, h = hs.shape
    nb = n // block
    body = functools.partial(_gate_body, seq, block, n_experts, bsz, const)
    idx, w8, ss = pl.pallas_call(
        body,
        grid=(nb,),
        in_specs=[
            pl.BlockSpec((block // 2, h), lambda i: (2 * i, 0)),
            pl.BlockSpec((block // 2, h), lambda i: (2 * i + 1, 0)),
            pl.BlockSpec((n_experts, h), lambda i: (0, 0)),
        ],
        out_specs=[
            pl.BlockSpec((block, TOP_K), lambda i: (i, 0)),
            pl.BlockSpec((block, TOP_K), lambda i: (i, 0)),
            pl.BlockSpec((bsz, n_experts), lambda i: (0, 0)),
        ],
        out_shape=[
            jax.ShapeDtypeStruct((n, TOP_K), jnp.int32),
            jax.ShapeDtypeStruct((n, TOP_K), jnp.float32),
            jax.ShapeDtypeStruct((bsz, n_experts), jnp.float32),
        ],
        scratch_shapes=[
            pltpu.VMEM((n_experts, 128), jnp.float32),
        ],
        interpret=interpret,
    )(hs, hs, gate_weight)
    return idx, w8, ss


def _gate(hidden_states, gate_weight, *, block=None, chunks=2,
          interpret=False):
    bsz, seq, h = hidden_states.shape
    n_experts = gate_weight.shape[0]
    n = bsz * seq
    if block is None:
        block = 2048 if seq % 2048 == 0 else seq
    hs = hidden_states.reshape(n, h)
    const = ALPHA / (bsz * (seq * TOP_K / n_experts) * seq)
    if bsz % chunks != 0:
        chunks = 1
    bsz_c = bsz // chunks
    n_c = n // chunks

    sc_aux = _make_sc_aux(n_c, n_experts, bsz_c, seq)
    idxs, wts, parts = [], [], []
    for c in range(chunks):
        idx_c, wt_c, ss_c = _tc_call(
            hs[c * n_c:(c + 1) * n_c], gate_weight, seq, block,
            n_experts, bsz_c, const, interpret=interpret)
        idxs.append(idx_c)
        wts.append(wt_c)
        parts.append(sc_aux(idx_c.reshape(n_c * TOP_K), ss_c))
    idx = jnp.concatenate(idxs, axis=0) if chunks > 1 else idxs[0]
    w8 = jnp.concatenate(wts, axis=0) if chunks > 1 else wts[0]
    aux = jnp.sum(jnp.stack(parts))
    return idx, w8, aux


def kernel(hidden_states, gate_weight):
    return _gate(hidden_states, gate_weight)


# speedup vs baseline: 2.0233x; 2.0233x over previous
import functools

import jax
import jax.numpy as jnp
from jax import lax
from jax.experimental import pallas as pl
from jax.experimental.pallas import tpu as pltpu
from jax.experimental.pallas import tpu_sc as plsc

TOP_K = 8
ALPHA = 0.001


def _gate_body(seq, block, n_experts, batch, const,
               hs_a_ref, hs_b_ref, w_ref, idx_ref, wt_ref, ss_ref, sc_acc):
    i = pl.program_id(0)
    nb = pl.num_programs(0)
    bpb = seq // block
    b = i // bpb

    @pl.when(i == 0)
    def _init():
        sc_acc[...] = jnp.zeros_like(sc_acc)

    w = w_ref[...]
    lt_a = jax.lax.dot_general(
        w, hs_a_ref[...], (((1,), (1,)), ((), ())),
        preferred_element_type=jnp.float32,
        precision=jax.lax.Precision.DEFAULT)
    lt_b = jax.lax.dot_general(
        w, hs_b_ref[...], (((1,), (1,)), ((), ())),
        preferred_element_type=jnp.float32,
        precision=jax.lax.Precision.DEFAULT)
    lt = jnp.concatenate([lt_a, lt_b], axis=1)

    m = jnp.max(lt, axis=0, keepdims=True)
    ex = jnp.exp(lt - m)
    s = jnp.sum(ex, axis=0, keepdims=True)

    rows = jax.lax.broadcasted_iota(jnp.int32, (n_experts, block), 0)
    rowsk = jax.lax.broadcasted_iota(jnp.int32, (TOP_K, block), 0)
    cur = lt
    mk8 = jnp.zeros((TOP_K, block), jnp.float32)
    ik8 = jnp.zeros((TOP_K, block), jnp.int32)
    for k in range(TOP_K):
        mk = jnp.max(cur, axis=0, keepdims=True)
        cand = jnp.where(cur == mk, rows, n_experts)
        ik = jnp.min(cand, axis=0, keepdims=True)
        mk8 = jnp.where(rowsk == k, mk, mk8)
        ik8 = jnp.where(rowsk == k, ik, ik8)
        cur = jnp.where(rows == ik, -jnp.inf, cur)
    w8 = jnp.exp(mk8 - m) / s
    idx_ref[...] = ik8.T
    wt_ref[...] = w8.T

    sc_local = jnp.sum(ex * (1.0 / s), axis=1, keepdims=True)
    lanes = jax.lax.broadcasted_iota(jnp.int32, sc_acc.shape, 1)
    bm = lanes == b
    sc_acc[...] = sc_acc[...] + jnp.where(bm, sc_local, 0.0)

    @pl.when(i == nb - 1)
    def _fin():
        ss_ref[...] = (sc_acc[...].T)[:batch, :] * const


def _make_sc_aux(n, n_experts, batch, seq):
    info = plsc.get_sparse_core_info()
    nw = info.num_cores * info.num_subcores
    per_w = n // nw
    per_wi = per_w * TOP_K
    chunks = per_wi // 16
    wpb = (batch * seq // per_w) // batch
    mesh = plsc.VectorSubcoreMesh(core_axis_name="c", subcore_axis_name="s")

    @functools.partial(
        pl.kernel, mesh=mesh,
        compiler_params=pltpu.CompilerParams(needs_layout_passes=False),
        out_type=jax.ShapeDtypeStruct((nw, 16), jnp.float32),
        scratch_types=[
            pltpu.VMEM((per_wi,), jnp.int32),
            pltpu.VMEM((2 * n_experts,), jnp.float32),
            pltpu.VMEM((n_experts,), jnp.float32),
            pltpu.VMEM((16,), jnp.float32),
        ],
    )
    def sc_aux(idx_hbm, ss_hbm, out_hbm, idx_v, cnt_v, row_v, acc_v):
        wid = lax.axis_index("s") * info.num_cores + lax.axis_index("c")
        b = wid // wpb
        pltpu.sync_copy(idx_hbm.at[pl.ds(wid * per_wi, per_wi)], idx_v)
        pltpu.sync_copy(ss_hbm.at[b], row_v)
        for j in range(2 * n_experts // 16):
            cnt_v[pl.ds(j * 16, 16)] = jnp.zeros((16,), jnp.float32)
        lane = jax.lax.broadcasted_iota(jnp.int32, (16,), 0)
        off = jnp.where(lane >= TOP_K, n_experts, 0)
        ones = jnp.ones((16,), jnp.float32)

        def body(j, _):
            v = idx_v[pl.ds(j * 16, 16)] + off
            plsc.addupdate_scatter(cnt_v, [v], ones)
            return 0

        lax.fori_loop(0, chunks, body, 0)
        acc = jnp.zeros((16,), jnp.float32)
        for j in range(2 * n_experts // 16):
            c = cnt_v[pl.ds(j * 16, 16)]
            r = row_v[pl.ds((j % (n_experts // 16)) * 16, 16)]
            acc = acc + c * r
        acc_v[...] = acc
        pltpu.sync_copy(acc_v, out_hbm.at[wid])

    return sc_aux


def _tc_call(hs, gate_weight, seq, block, n_experts, bsz, const,
             n, base_blk=0, interpret=False):
    h = hs.shape[1]
    nb = n // block
    body = functools.partial(_gate_body, seq, block, n_experts, bsz, const)
    idx, w8, ss = pl.pallas_call(
        body,
        grid=(nb,),
        in_specs=[
            pl.BlockSpec((block // 2, h),
                         lambda i, b=base_blk: (2 * (b + i), 0)),
            pl.BlockSpec((block // 2, h),
                         lambda i, b=base_blk: (2 * (b + i) + 1, 0)),
            pl.BlockSpec((n_experts, h), lambda i: (0, 0)),
        ],
        out_specs=[
            pl.BlockSpec((block, TOP_K), lambda i: (i, 0)),
            pl.BlockSpec((block, TOP_K), lambda i: (i, 0)),
            pl.BlockSpec((bsz, n_experts), lambda i: (0, 0)),
        ],
        out_shape=[
            jax.ShapeDtypeStruct((n, TOP_K), jnp.int32),
            jax.ShapeDtypeStruct((n, TOP_K), jnp.float32),
            jax.ShapeDtypeStruct((bsz, n_experts), jnp.float32),
        ],
        scratch_shapes=[
            pltpu.VMEM((n_experts, 128), jnp.float32),
        ],
        interpret=interpret,
    )(hs, hs, gate_weight)
    return idx, w8, ss


def _gate(hidden_states, gate_weight, *, block=None, chunks=2,
          interpret=False):
    bsz, seq, h = hidden_states.shape
    n_experts = gate_weight.shape[0]
    n = bsz * seq
    if block is None:
        block = 2048 if seq % 2048 == 0 else seq
    hs = hidden_states.reshape(n, h)
    const = ALPHA / (bsz * (seq * TOP_K / n_experts) * seq)
    if bsz % chunks != 0:
        chunks = 1
    bsz_c = bsz // chunks
    n_c = n // chunks

    sc_aux = _make_sc_aux(n_c, n_experts, bsz_c, seq)
    idxs, wts, parts = [], [], []
    for c in range(chunks):
        idx_c, wt_c, ss_c = _tc_call(
            hs, gate_weight, seq, block, n_experts, bsz_c, const,
            n_c, base_blk=c * (n_c // block), interpret=interpret)
        idxs.append(idx_c)
        wts.append(wt_c)
        parts.append(sc_aux(idx_c.reshape(n_c * TOP_K), ss_c))
    idx = jnp.concatenate(idxs, axis=0) if chunks > 1 else idxs[0]
    w8 = jnp.concatenate(wts, axis=0) if chunks > 1 else wts[0]
    aux = jnp.sum(jnp.stack(parts))
    return idx, w8, aux


def kernel(hidden_states, gate_weight):
    return _gate(hidden_states, gate_weight)


# speedup vs baseline: 2.8411x; 1.4042x over previous
import functools

import jax
import jax.numpy as jnp
from jax.experimental import pallas as pl
from jax.experimental.pallas import tpu as pltpu

TOP_K = 8
ALPHA = 0.001


def _gate_body(seq, block, n_experts, batch,
               hs_a_ref, hs_b_ref, w_ref, idx_ref, wt_ref, aux_ref,
               cnt_acc, sc_acc):
    i = pl.program_id(0)
    nb = pl.num_programs(0)
    bpb = seq // block
    b = i // bpb

    @pl.when(i == 0)
    def _init():
        cnt_acc[...] = jnp.zeros_like(cnt_acc)
        sc_acc[...] = jnp.zeros_like(sc_acc)

    w = w_ref[...]
    lt_a = jax.lax.dot_general(
        w, hs_a_ref[...], (((1,), (1,)), ((), ())),
        preferred_element_type=jnp.float32,
        precision=jax.lax.Precision.DEFAULT)
    lt_b = jax.lax.dot_general(
        w, hs_b_ref[...], (((1,), (1,)), ((), ())),
        preferred_element_type=jnp.float32,
        precision=jax.lax.Precision.DEFAULT)
    lt = jnp.concatenate([lt_a, lt_b], axis=1)

    m = jnp.max(lt, axis=0, keepdims=True)
    ex = jnp.exp(lt - m)
    s = jnp.sum(ex, axis=0, keepdims=True)

    rows = jax.lax.broadcasted_iota(jnp.int32, (n_experts, block), 0)
    rowsk = jax.lax.broadcasted_iota(jnp.int32, (TOP_K, block), 0)
    cur = lt
    mk8 = jnp.zeros((TOP_K, block), jnp.float32)
    ik8 = jnp.zeros((TOP_K, block), jnp.int32)
    for k in range(TOP_K):
        mk = jnp.max(cur, axis=0, keepdims=True)
        cand = jnp.where(cur == mk, rows, n_experts)
        ik = jnp.min(cand, axis=0, keepdims=True)
        mk8 = jnp.where(rowsk == k, mk, mk8)
        ik8 = jnp.where(rowsk == k, ik, ik8)
        cur = jnp.where(rows == ik, -jnp.inf, cur)
    w8 = jnp.exp(mk8 - m) / s
    idx_ref[...] = ik8.T
    wt_ref[...] = w8.T

    sel = (cur < jnp.float32(-3e38)).astype(jnp.float32)
    cnt_local = jnp.sum(sel, axis=1, keepdims=True)
    sc_local = jnp.sum(ex * (1.0 / s), axis=1, keepdims=True)
    lanes = jax.lax.broadcasted_iota(jnp.int32, cnt_acc.shape, 1)
    bm = lanes == b
    cnt_acc[...] = cnt_acc[...] + jnp.where(bm, cnt_local, 0.0)
    sc_acc[...] = sc_acc[...] + jnp.where(bm, sc_local, 0.0)

    @pl.when(i == nb - 1)
    def _fin():
        const = ALPHA / (batch * (seq * TOP_K / n_experts) * seq)
        aux_ref[...] = jnp.sum(cnt_acc[...] * sc_acc[...],
                               axis=(0, 1), keepdims=True) * const


def _gate(hidden_states, gate_weight, *, block=None, interpret=False):
    bsz, seq, h = hidden_states.shape
    n_experts = gate_weight.shape[0]
    n = bsz * seq
    if block is None:
        block = 2048 if seq % 2048 == 0 else seq
    hs = hidden_states.reshape(n, h)
    nb = n // block

    body = functools.partial(_gate_body, seq, block, n_experts, bsz)
    idx, w8, aux = pl.pallas_call(
        body,
        grid=(nb,),
        in_specs=[
            pl.BlockSpec((block // 2, h), lambda i: (2 * i, 0)),
            pl.BlockSpec((block // 2, h), lambda i: (2 * i + 1, 0)),
            pl.BlockSpec((n_experts, h), lambda i: (0, 0)),
        ],
        out_specs=[
            pl.BlockSpec((block, TOP_K), lambda i: (i, 0)),
            pl.BlockSpec((block, TOP_K), lambda i: (i, 0)),
            pl.BlockSpec((1, 1), lambda i: (0, 0)),
        ],
        out_shape=[
            jax.ShapeDtypeStruct((n, TOP_K), jnp.int32),
            jax.ShapeDtypeStruct((n, TOP_K), jnp.float32),
            jax.ShapeDtypeStruct((1, 1), jnp.float32),
        ],
        scratch_shapes=[
            pltpu.VMEM((n_experts, 128), jnp.float32),
            pltpu.VMEM((n_experts, 128), jnp.float32),
        ],
        interpret=interpret,
    )(hs, hs, gate_weight)
    return idx, w8, aux.reshape(())


def kernel(hidden_states, gate_weight):
    return _gate(hidden_states, gate_weight)
